# baseline (speedup 1.0000x reference)
import functools

import jax
import jax.numpy as jnp
from jax import lax
from jax.experimental import pallas as pl
from jax.experimental.pallas import tpu as pltpu
from jax.experimental.pallas import tpu_sc as plsc

SEQ_LEN = 4096
N_EMBD = 2048


def _make_copy():
    try:
        info = plsc.get_sparse_core_info()
        num_cores, num_subcores = info.num_cores, info.num_subcores
    except Exception:
        num_cores, num_subcores = 2, 16
    num_workers = num_cores * num_subcores
    rows_per = SEQ_LEN // num_workers
    mesh = plsc.VectorSubcoreMesh(core_axis_name="c", subcore_axis_name="s")

    @functools.partial(
        pl.kernel,
        mesh=mesh,
        out_type=jax.ShapeDtypeStruct((SEQ_LEN, N_EMBD), jnp.float32),
    )
    def copy_k(emb_hbm, out_hbm):
        wid = lax.axis_index("s") * num_cores + lax.axis_index("c")
        base = wid * rows_per
        pltpu.sync_copy(
            emb_hbm.at[pl.ds(base, rows_per)],
            out_hbm.at[pl.ds(base, rows_per)],
        )

    return copy_k


_copy = _make_copy()


def kernel(x, emb_weight):
    del x
    return _copy(emb_weight)


# speedup vs baseline: 24.9305x; 24.9305x over previous
import functools

import jax
import jax.numpy as jnp
from jax import lax
from jax.experimental import pallas as pl
from jax.experimental.pallas import tpu as pltpu
from jax.experimental.pallas import tpu_sc as plsc

SEQ_LEN = 4096
N_EMBD = 2048


def _make_copy():
    try:
        info = plsc.get_sparse_core_info()
        num_cores, num_subcores = info.num_cores, info.num_subcores
    except Exception:
        num_cores, num_subcores = 2, 16
    num_workers = num_cores * num_subcores
    rows_per = SEQ_LEN // num_workers
    mesh = plsc.VectorSubcoreMesh(core_axis_name="c", subcore_axis_name="s")

    chunk = 16
    nchunks = rows_per // chunk
    nbuf = 3

    @functools.partial(
        pl.kernel,
        mesh=mesh,
        out_type=jax.ShapeDtypeStruct((SEQ_LEN, N_EMBD), jnp.float32),
        scratch_types=(
            [pltpu.VMEM((nbuf, chunk, N_EMBD), jnp.float32)]
            + [pltpu.SemaphoreType.DMA] * (2 * nbuf)
        ),
    )
    def copy_k(emb_hbm, out_hbm, buf, *sems):
        sem_in, sem_out = sems[:nbuf], sems[nbuf:]
        wid = lax.axis_index("s") * num_cores + lax.axis_index("c")
        base = wid * rows_per

        def cp_in(c):
            b = c % nbuf
            return pltpu.async_copy(
                emb_hbm.at[pl.ds(base + c * chunk, chunk)], buf.at[b], sem_in[b])

        def cp_out(c):
            b = c % nbuf
            return pltpu.async_copy(
                buf.at[b], out_hbm.at[pl.ds(base + c * chunk, chunk)], sem_out[b])

        ins, outs = {}, {}
        for c in range(min(nbuf, nchunks)):
            ins[c] = cp_in(c)
        for c in range(nchunks):
            ins[c].wait()
            outs[c] = cp_out(c)
            nxt = c + nbuf
            if nxt < nchunks:
                outs[c].wait()
                ins[nxt] = cp_in(nxt)
        for c in range(max(0, nchunks - nbuf), nchunks):
            outs[c].wait()

    return copy_k


_copy = _make_copy()


def kernel(x, emb_weight):
    del x
    return _copy(emb_weight)
